# baseline (speedup 1.0000x reference)
import functools

import jax
import jax.numpy as jnp
from jax import lax
from jax.experimental import pallas as pl
from jax.experimental.pallas import tpu as pltpu
from jax.experimental.pallas import tpu_sc as plsc

B = 16384
D = 32
W = 128
PACK = W // D
L = 16
NC = 2
NS = 16
NW = NC * NS
BPW = B // NW
CHUNK = 128
NCHUNK = BPW // CHUNK
GPC = CHUNK // L


def _body(user_hbm, item_hbm, eu_hbm, ei_hbm, bu_hbm, bi_hbm, out_hbm,
          uidx, iidx, urow, irow,
          tu0, tu1, tv0, tv1, bu0, bu1, bi0, bi1, outv, s0, s1):
    bufs = [(tu0, tv0, bu0, bi0, s0), (tu1, tv1, bu1, bi1, s1)]
    wid = lax.axis_index("s") * NC + lax.axis_index("c")
    base = wid * BPW

    for j in range(NCHUNK):
        pltpu.sync_copy(user_hbm.at[pl.ds(base + j * CHUNK, CHUNK)],
                        uidx.at[j])
        pltpu.sync_copy(item_hbm.at[pl.ds(base + j * CHUNK, CHUNK)],
                        iidx.at[j])
        for g in range(GPC):
            sl = pl.ds(g * L, L)
            urow[j, sl] = uidx[j, sl] >> 2
            irow[j, sl] = iidx[j, sl] >> 2

    def fire(j):
        tu, tv, bu, bi, sem = bufs[j & 1]
        return (
            pltpu.async_copy(eu_hbm.at[urow.at[j]], tu, sem),
            pltpu.async_copy(ei_hbm.at[irow.at[j]], tv, sem),
            pltpu.async_copy(bu_hbm.at[uidx.at[j]], bu, sem),
            pltpu.async_copy(bi_hbm.at[iidx.at[j]], bi, sem),
        )

    iota = lax.iota(jnp.int32, L)
    inflight = fire(0)

    for j in range(NCHUNK):
        cur = inflight
        if j + 1 < NCHUNK:
            inflight = fire(j + 1)
        for c in cur:
            c.wait()
        tu, tv, bu, bi, _ = bufs[j & 1]

        def group(g, _, j=j, tu=tu, tv=tv, bu=bu, bi=bi):
            sl = pl.ds(pl.multiple_of(g * L, L), L)
            row = g * L + iota
            uvec = uidx[j, sl]
            ivec = iidx[j, sl]
            subu = (uvec & (PACK - 1)) << 5
            subv = (ivec & (PACK - 1)) << 5
            accs = [jnp.zeros((L,), jnp.float32) for _ in range(4)]
            rot = iota
            for d in range(D):
                uval = plsc.load_gather(tu, [row, subu + rot])
                vval = plsc.load_gather(tv, [row, subv + rot])
                accs[d & 3] = accs[d & 3] + uval * vval
                rot = (rot + 1) & (D - 1)
            acc = ((accs[0] + accs[1]) + (accs[2] + accs[3])
                   + bu[sl] + bi[sl] + 3.5)
            acc = jnp.minimum(jnp.maximum(acc, 0.0), 5.0)
            outv[pl.ds(pl.multiple_of(j * CHUNK + g * L, L), L)] = acc
            return 0

        lax.fori_loop(0, GPC, group, 0)

    pltpu.sync_copy(outv, out_hbm.at[pl.ds(base, BPW)])


@jax.jit
def _bpr(user, item, embed_user, embed_item, bias_user, bias_item):
    mesh = plsc.VectorSubcoreMesh(core_axis_name="c", subcore_axis_name="s",
                                  num_cores=NC, num_subcores=NS)
    run = functools.partial(
        pl.kernel,
        out_type=jax.ShapeDtypeStruct((B,), jnp.float32),
        mesh=mesh,
        compiler_params=pltpu.CompilerParams(needs_layout_passes=False),
        scratch_types=[
            pltpu.VMEM((NCHUNK, CHUNK), jnp.int32),
            pltpu.VMEM((NCHUNK, CHUNK), jnp.int32),
            pltpu.VMEM((NCHUNK, CHUNK), jnp.int32),
            pltpu.VMEM((NCHUNK, CHUNK), jnp.int32),
            pltpu.VMEM((CHUNK, W), jnp.float32),
            pltpu.VMEM((CHUNK, W), jnp.float32),
            pltpu.VMEM((CHUNK, W), jnp.float32),
            pltpu.VMEM((CHUNK, W), jnp.float32),
            pltpu.VMEM((CHUNK,), jnp.float32),
            pltpu.VMEM((CHUNK,), jnp.float32),
            pltpu.VMEM((CHUNK,), jnp.float32),
            pltpu.VMEM((CHUNK,), jnp.float32),
            pltpu.VMEM((BPW,), jnp.float32),
            pltpu.SemaphoreType.DMA,
            pltpu.SemaphoreType.DMA,
        ],
    )(_body)
    return run(user, item,
               embed_user.reshape(-1, W), embed_item.reshape(-1, W),
               bias_user.reshape(-1), bias_item.reshape(-1))


def kernel(user, item, embed_user, embed_item, bias_user, bias_item):
    return _bpr(user, item, embed_user, embed_item, bias_user, bias_item)


# speedup vs baseline: 1.0005x; 1.0005x over previous
import functools

import jax
import jax.numpy as jnp
from jax import lax
from jax.experimental import pallas as pl
from jax.experimental.pallas import tpu as pltpu
from jax.experimental.pallas import tpu_sc as plsc

B = 16384
D = 32
W = 128
PACK = W // D
L = 16
NC = 2
NS = 16
NW = NC * NS
BPW = B // NW
CHUNK = 128
NCHUNK = BPW // CHUNK
GPC = CHUNK // L


def _body(user_hbm, item_hbm, eu_hbm, ei_hbm, buT_hbm, biT_hbm, out_hbm,
          uidx, iidx, urow, irow,
          tu0, tu1, tv0, tv1, bu0, bu1, bi0, bi1, outv, s0, s1):
    bufs = [(tu0, tv0, bu0, bi0, s0), (tu1, tv1, bu1, bi1, s1)]
    sub = lax.axis_index("s")
    wid = sub * NC + lax.axis_index("c")
    base = wid * BPW

    for j in range(NCHUNK):
        pltpu.sync_copy(user_hbm.at[pl.ds(base + j * CHUNK, CHUNK)],
                        uidx.at[j])
        pltpu.sync_copy(item_hbm.at[pl.ds(base + j * CHUNK, CHUNK)],
                        iidx.at[j])
        for g in range(GPC):
            sl = pl.ds(g * L, L)
            urow[j, sl] = uidx[j, sl] >> 2
            irow[j, sl] = iidx[j, sl] >> 2

    def fire(j):
        tu, tv, bu, bi, sem = bufs[j & 1]
        return (
            pltpu.async_copy(eu_hbm.at[urow.at[j]], tu, sem),
            pltpu.async_copy(ei_hbm.at[irow.at[j]], tv, sem),
            pltpu.async_copy(buT_hbm.at[0].at[uidx.at[j]], bu, sem),
            pltpu.async_copy(biT_hbm.at[0].at[iidx.at[j]], bi, sem),
        )

    iota = lax.iota(jnp.int32, L)
    inflight = fire(0)

    for j in range(NCHUNK):
        cur = inflight
        if j + 1 < NCHUNK:
            inflight = fire(j + 1)
        for c in cur:
            c.wait()
        tu, tv, bu, bi, _ = bufs[j & 1]

        def group(g, _, j=j, tu=tu, tv=tv, bu=bu, bi=bi):
            sl = pl.ds(pl.multiple_of(g * L, L), L)
            row = g * L + iota
            uvec = uidx[j, sl]
            ivec = iidx[j, sl]
            subu = (uvec & (PACK - 1)) << 5
            subv = (ivec & (PACK - 1)) << 5
            accs = [jnp.zeros((L,), jnp.float32) for _ in range(4)]
            rot = iota
            for d in range(D):
                uval = plsc.load_gather(tu, [row, subu + rot])
                vval = plsc.load_gather(tv, [row, subv + rot])
                accs[d & 3] = accs[d & 3] + uval * vval
                rot = (rot + 1) & (D - 1)
            acc = ((accs[0] + accs[1]) + (accs[2] + accs[3])
                   + bu[sl] + bi[sl] + 3.5)
            acc = jnp.minimum(jnp.maximum(acc, 0.0), 5.0)
            outv[pl.ds(pl.multiple_of(j * CHUNK + g * L, L), L)] = acc
            return 0

        lax.fori_loop(0, GPC, group, 0)

    pltpu.sync_copy(outv, out_hbm.at[pl.ds(base, BPW)])


@jax.jit
def _bpr(user, item, embed_user, embed_item, bias_user, bias_item):
    mesh = plsc.VectorSubcoreMesh(core_axis_name="c", subcore_axis_name="s",
                                  num_cores=NC, num_subcores=NS)
    run = functools.partial(
        pl.kernel,
        out_type=jax.ShapeDtypeStruct((B,), jnp.float32),
        mesh=mesh,
        compiler_params=pltpu.CompilerParams(needs_layout_passes=False),
        scratch_types=[
            pltpu.VMEM((NCHUNK, CHUNK), jnp.int32),
            pltpu.VMEM((NCHUNK, CHUNK), jnp.int32),
            pltpu.VMEM((NCHUNK, CHUNK), jnp.int32),
            pltpu.VMEM((NCHUNK, CHUNK), jnp.int32),
            pltpu.VMEM((CHUNK, W), jnp.float32),
            pltpu.VMEM((CHUNK, W), jnp.float32),
            pltpu.VMEM((CHUNK, W), jnp.float32),
            pltpu.VMEM((CHUNK, W), jnp.float32),
            pltpu.VMEM((CHUNK,), jnp.float32),
            pltpu.VMEM((CHUNK,), jnp.float32),
            pltpu.VMEM((CHUNK,), jnp.float32),
            pltpu.VMEM((CHUNK,), jnp.float32),
            pltpu.VMEM((BPW,), jnp.float32),
            pltpu.SemaphoreType.DMA,
            pltpu.SemaphoreType.DMA,
        ],
    )(_body)
    return run(user, item,
               embed_user.reshape(-1, W), embed_item.reshape(-1, W),
               bias_user.T, bias_item.T)


def kernel(user, item, embed_user, embed_item, bias_user, bias_item):
    return _bpr(user, item, embed_user, embed_item, bias_user, bias_item)


# speedup vs baseline: 1.0967x; 1.0961x over previous
import functools

import jax
import jax.numpy as jnp
from jax import lax
from jax.experimental import pallas as pl
from jax.experimental.pallas import tpu as pltpu
from jax.experimental.pallas import tpu_sc as plsc

B = 16384
D = 32
W = 128
PACK = W // D
L = 16
NC = 2
NS = 16
NW = NC * NS
BPW = B // NW
CHUNK = 128
NCHUNK = BPW // CHUNK
GPC = CHUNK // L


def _body(user_hbm, item_hbm, eu_hbm, ei_hbm, buT_hbm, biT_hbm, out_hbm,
          uidx, iidx, urow, irow,
          tu0, tu1, tv0, tv1, bu0, bu1, bi0, bi1, outv, s0, s1):
    bufs = [(tu0, tv0, bu0, bi0, s0), (tu1, tv1, bu1, bi1, s1)]
    sub = lax.axis_index("s")
    wid = sub * NC + lax.axis_index("c")
    base = wid * BPW

    for j in range(NCHUNK):
        pltpu.sync_copy(user_hbm.at[pl.ds(base + j * CHUNK, CHUNK)],
                        uidx.at[j])
        pltpu.sync_copy(item_hbm.at[pl.ds(base + j * CHUNK, CHUNK)],
                        iidx.at[j])
        for g in range(GPC):
            sl = pl.ds(g * L, L)
            urow[j, sl] = uidx[j, sl] >> 2
            irow[j, sl] = iidx[j, sl] >> 2

    def fire(j):
        tu, tv, bu, bi, sem = bufs[j & 1]
        return (
            pltpu.async_copy(eu_hbm.at[urow.at[j]], tu, sem),
            pltpu.async_copy(ei_hbm.at[irow.at[j]], tv, sem),
            pltpu.async_copy(buT_hbm.at[0].at[uidx.at[j]], bu, sem),
            pltpu.async_copy(biT_hbm.at[0].at[iidx.at[j]], bi, sem),
        )

    iota = lax.iota(jnp.int32, L)
    inflight = fire(0)

    for j in range(NCHUNK):
        cur = inflight
        if j + 1 < NCHUNK:
            inflight = fire(j + 1)
        for c in cur:
            c.wait()
        tu, tv, bu, bi, _ = bufs[j & 1]

        def group(g, _, j=j, tu=tu, tv=tv, bu=bu, bi=bi):
            sl = pl.ds(pl.multiple_of(g * L, L), L)
            row = g * L + iota
            uvec = uidx[j, sl]
            ivec = iidx[j, sl]
            subu = (uvec & (PACK - 1)) << 5
            subv = (ivec & (PACK - 1)) << 5
            accs = [jnp.zeros((L,), jnp.float32) for _ in range(4)]
            rot = iota
            for d in range(D):
                uval = plsc.load_gather(tu, [row, subu + rot])
                vval = plsc.load_gather(tv, [row, subv + rot])
                accs[d & 3] = accs[d & 3] + uval * vval
                rot = (rot + 1) & (D - 1)
            acc = ((accs[0] + accs[1]) + (accs[2] + accs[3])
                   + bu[sl] + bi[sl] + 3.5)
            acc = jnp.minimum(jnp.maximum(acc, 0.0), 5.0)
            outv[pl.ds(pl.multiple_of(j * CHUNK + g * L, L), L)] = acc
            return 0

        lax.fori_loop(0, GPC, group, 0)

    pltpu.sync_copy(outv, out_hbm.at[pl.ds(base, BPW)])


@jax.jit
def _bpr(user, item, embed_user, embed_item, bias_user, bias_item):
    mesh = plsc.VectorSubcoreMesh(core_axis_name="c", subcore_axis_name="s",
                                  num_cores=NC, num_subcores=NS)
    run = functools.partial(
        pl.kernel,
        out_type=jax.ShapeDtypeStruct((B,), jnp.float32),
        mesh=mesh,
        compiler_params=pltpu.CompilerParams(needs_layout_passes=False),
        scratch_types=[
            pltpu.VMEM((NCHUNK, CHUNK), jnp.int32),
            pltpu.VMEM((NCHUNK, CHUNK), jnp.int32),
            pltpu.VMEM((NCHUNK, CHUNK), jnp.int32),
            pltpu.VMEM((NCHUNK, CHUNK), jnp.int32),
            pltpu.VMEM((CHUNK, W), jnp.float32),
            pltpu.VMEM((CHUNK, W), jnp.float32),
            pltpu.VMEM((CHUNK, W), jnp.float32),
            pltpu.VMEM((CHUNK, W), jnp.float32),
            pltpu.VMEM((CHUNK,), jnp.float32),
            pltpu.VMEM((CHUNK,), jnp.float32),
            pltpu.VMEM((CHUNK,), jnp.float32),
            pltpu.VMEM((CHUNK,), jnp.float32),
            pltpu.VMEM((BPW,), jnp.float32),
            pltpu.SemaphoreType.DMA,
            pltpu.SemaphoreType.DMA,
        ],
    )(_body)
    eu = jnp.transpose(embed_user.T.reshape(D, -1, PACK), (1, 2, 0))
    ei = jnp.transpose(embed_item.T.reshape(D, -1, PACK), (1, 2, 0))
    return run(user, item,
               eu.reshape(-1, W), ei.reshape(-1, W),
               bias_user.T, bias_item.T)


def kernel(user, item, embed_user, embed_item, bias_user, bias_item):
    return _bpr(user, item, embed_user, embed_item, bias_user, bias_item)
